# baseline (speedup 1.0000x reference)
import jax
import jax.numpy as jnp
from jax import lax
from jax.experimental import pallas as pl
from jax.experimental.pallas import tpu as pltpu
from jax.experimental.pallas import tpu_sc as plsc

DIM = 64
NW = 32
CHUNK = 512
NBUF = 2
AHEAD = 1


def _emb_body(table_hbm, x_hbm, out_hbm, idx_v, *rest):
    rows = rest[:NBUF]
    gsem = rest[NBUF:2 * NBUF]
    osem = rest[2 * NBUF:3 * NBUF]

    n_chunks = x_hbm.shape[0] // NW
    wid = lax.axis_index("s") * 2 + lax.axis_index("c")
    base_row = wid * n_chunks * CHUNK

    pltpu.sync_copy(x_hbm.at[pl.ds(wid * n_chunks, n_chunks)], idx_v)

    def fire_gather(g, b):
        return pltpu.async_copy(table_hbm.at[idx_v.at[g]], rows[b], gsem[b])

    def fire_out(g, b):
        dst = out_hbm.at[pl.ds(base_row + g * CHUNK, CHUNK)]
        return pltpu.async_copy(rows[b], dst, osem[b])

    for f in range(AHEAD):
        fire_gather(f, f % NBUF)

    def outer(i, carry):
        g0 = i * NBUF
        for b in range(NBUF):
            g = g0 + b
            f = g + AHEAD
            bf = (b + AHEAD) % NBUF

            @pl.when(f < n_chunks)
            def _():
                @pl.when(f >= NBUF)
                def _():
                    pltpu.make_async_copy(
                        rows[bf],
                        out_hbm.at[pl.ds(base_row, CHUNK)],
                        osem[bf],
                    ).wait()
                fire_gather(f, bf)

            pltpu.make_async_copy(
                table_hbm.at[idx_v.at[g]], rows[b], gsem[b]
            ).wait()
            fire_out(g, b)
        return carry

    lax.fori_loop(0, n_chunks // NBUF, outer, 0)

    for b in range(NBUF):
        pltpu.make_async_copy(
            rows[b], out_hbm.at[pl.ds(base_row, CHUNK)], osem[b]
        ).wait()


def kernel(x, table):
    B, L = x.shape
    n_total = B * L
    n_chunks_total = n_total // CHUNK
    xf = x.reshape(n_chunks_total, CHUNK).astype(jnp.int32)
    scratch = (
        [pltpu.VMEM((n_chunks_total // NW, CHUNK), jnp.int32)]
        + [pltpu.VMEM((CHUNK, DIM), jnp.float32) for _ in range(NBUF)]
        + [pltpu.SemaphoreType.DMA for _ in range(2 * NBUF)]
    )
    k = pl.kernel(
        _emb_body,
        out_type=jax.ShapeDtypeStruct((n_total, DIM), jnp.float32),
        mesh=plsc.VectorSubcoreMesh(core_axis_name="c", subcore_axis_name="s"),
        scratch_types=scratch,
        compiler_params=pltpu.CompilerParams(use_tc_tiling_on_sc=False),
    )
    out = k(table, xf)
    return out.reshape(B, L, DIM)


# speedup vs baseline: 1.0048x; 1.0048x over previous
import jax
import jax.numpy as jnp
from jax import lax
from jax.experimental import pallas as pl
from jax.experimental.pallas import tpu as pltpu
from jax.experimental.pallas import tpu_sc as plsc

DIM = 64
NW = 32
NBUF = 4
AHEAD = 2


def _emb_body(table_hbm, x_hbm, out_hbm, idx_v, *rest):
    rows = rest[:NBUF]
    gsem = rest[NBUF:2 * NBUF]
    osem = rest[2 * NBUF:3 * NBUF]

    B, L = x_hbm.shape
    n_chunks = B // NW
    wid = lax.axis_index("s") * 2 + lax.axis_index("c")
    base = wid * n_chunks

    pltpu.sync_copy(x_hbm.at[pl.ds(base, n_chunks)], idx_v)

    def fire_gather(g, b):
        return pltpu.async_copy(table_hbm.at[idx_v.at[g]], rows[b], gsem[b])

    def fire_out(g, b):
        return pltpu.async_copy(rows[b], out_hbm.at[base + g], osem[b])

    for f in range(AHEAD):
        fire_gather(f, f % NBUF)

    def outer(i, carry):
        g0 = i * NBUF
        for b in range(NBUF):
            g = g0 + b
            f = g + AHEAD
            bf = (b + AHEAD) % NBUF

            @pl.when(f < n_chunks)
            def _():
                @pl.when(f >= NBUF)
                def _():
                    pltpu.make_async_copy(
                        rows[bf], out_hbm.at[base], osem[bf]
                    ).wait()
                fire_gather(f, bf)

            pltpu.make_async_copy(
                table_hbm.at[idx_v.at[g]], rows[b], gsem[b]
            ).wait()
            fire_out(g, b)
        return carry

    lax.fori_loop(0, n_chunks // NBUF, outer, 0)

    for b in range(NBUF):
        pltpu.make_async_copy(rows[b], out_hbm.at[base], osem[b]).wait()


def kernel(x, table):
    B, L = x.shape
    xi = x.astype(jnp.int32)
    scratch = (
        [pltpu.VMEM((B // NW, L), jnp.int32)]
        + [pltpu.VMEM((L, DIM), jnp.float32) for _ in range(NBUF)]
        + [pltpu.SemaphoreType.DMA for _ in range(2 * NBUF)]
    )
    k = pl.kernel(
        _emb_body,
        out_type=jax.ShapeDtypeStruct((B, L, DIM), jnp.float32),
        mesh=plsc.VectorSubcoreMesh(core_axis_name="c", subcore_axis_name="s"),
        scratch_types=scratch,
        compiler_params=pltpu.CompilerParams(use_tc_tiling_on_sc=False),
    )
    return k(table, xi)


# speedup vs baseline: 1.2235x; 1.2176x over previous
import jax
import jax.numpy as jnp
from jax import lax
from jax.experimental import pallas as pl
from jax.experimental.pallas import tpu as pltpu
from jax.experimental.pallas import tpu_sc as plsc

DIM = 64
PADDIM = 128
NW = 32
NBUF = 4
AHEAD = 2


def _emb_body(table_hbm, x_hbm, out_hbm, idx_v, *rest):
    rows = rest[:NBUF]
    gsem = rest[NBUF:2 * NBUF]
    osem = rest[2 * NBUF:3 * NBUF]

    NC, CHUNK = x_hbm.shape
    n_chunks = NC // NW
    wid = lax.axis_index("s") * 2 + lax.axis_index("c")
    base = wid * n_chunks

    pltpu.sync_copy(x_hbm.at[pl.ds(base, n_chunks)], idx_v)

    def fire_gather(g, b):
        return pltpu.async_copy(table_hbm.at[idx_v.at[g]], rows[b], gsem[b])

    def fire_out(g, b):
        dst = out_hbm.at[pl.ds((base + g) * CHUNK, CHUNK)]
        return pltpu.async_copy(rows[b], dst, osem[b])

    for f in range(AHEAD):
        fire_gather(f, f % NBUF)

    def outer(i, carry):
        g0 = i * NBUF
        for b in range(NBUF):
            g = g0 + b
            f = g + AHEAD
            bf = (b + AHEAD) % NBUF

            @pl.when(f < n_chunks)
            def _():
                @pl.when(f >= NBUF)
                def _():
                    pltpu.make_async_copy(
                        rows[bf],
                        out_hbm.at[pl.ds(base * CHUNK, CHUNK)],
                        osem[bf],
                    ).wait()
                fire_gather(f, bf)

            pltpu.make_async_copy(
                table_hbm.at[idx_v.at[g]], rows[b], gsem[b]
            ).wait()
            fire_out(g, b)
        return carry

    lax.fori_loop(0, n_chunks // NBUF, outer, 0)

    for b in range(NBUF):
        pltpu.make_async_copy(
            rows[b],
            out_hbm.at[pl.ds(base * CHUNK, CHUNK)],
            osem[b],
        ).wait()


CHUNK = 128


def kernel(x, table):
    B, L = x.shape
    n_total = B * L
    xi = x.reshape(n_total // CHUNK, CHUNK).astype(jnp.int32)
    tpad = jnp.pad(table, ((0, 0), (0, PADDIM - DIM)))
    scratch = (
        [pltpu.VMEM((n_total // CHUNK // NW, CHUNK), jnp.int32)]
        + [pltpu.VMEM((CHUNK, PADDIM), jnp.float32) for _ in range(NBUF)]
        + [pltpu.SemaphoreType.DMA for _ in range(2 * NBUF)]
    )
    k = pl.kernel(
        _emb_body,
        out_type=jax.ShapeDtypeStruct((n_total, PADDIM), jnp.float32),
        mesh=plsc.VectorSubcoreMesh(core_axis_name="c", subcore_axis_name="s"),
        scratch_types=scratch,
        compiler_params=pltpu.CompilerParams(use_tc_tiling_on_sc=True),
    )
    return k(tpad, xi).reshape(B, L, PADDIM)[..., :DIM]
